# Initial kernel scaffold; baseline (speedup 1.0000x reference)
import jax
import jax.numpy as jnp
from jax.experimental import pallas as pl


def kernel(pos_s, pos_e, pe_table, W, b):
    raise NotImplementedError("write your pallas kernel here")


# speedup vs baseline: 3.2683x; 3.2683x over previous
import functools

import jax
import jax.numpy as jnp
from jax import lax
from jax.experimental import pallas as pl
from jax.experimental.pallas import tpu as pltpu
from jax.experimental.pallas import tpu_sc as plsc

_NC, _NS = 2, 16
_NW = _NC * _NS
_CHUNK = 80
_LANES = 16


def _fuse_tables_body(pe_ref, a_ref, b_ref, out_ref):
    out_ref[0] = jnp.dot(pe_ref[...], a_ref[0],
                         preferred_element_type=jnp.float32) + b_ref[...]


def _make_tables(pe_table, W, b):
    P, H = pe_table.shape
    A = W.reshape(H, 4, H).transpose(1, 2, 0)
    bq = (0.25 * b).reshape(1, H).astype(jnp.float32)
    T = pl.pallas_call(
        _fuse_tables_body,
        grid=(4,),
        in_specs=[
            pl.BlockSpec((P, H), lambda k: (0, 0)),
            pl.BlockSpec((1, H, H), lambda k: (k, 0, 0)),
            pl.BlockSpec((1, H), lambda k: (0, 0)),
        ],
        out_specs=pl.BlockSpec((1, P, H), lambda k: (k, 0, 0)),
        out_shape=jax.ShapeDtypeStruct((4, P, H), jnp.float32),
    )(pe_table.astype(jnp.float32), A.astype(jnp.float32), bq)
    return T.reshape(4 * P, H)


def _make_sc_lookup(n_rows, H):
    per_w = n_rows // _NW
    nch = per_w // _CHUNK
    vec = H // _LANES
    mesh = plsc.VectorSubcoreMesh(core_axis_name="c", subcore_axis_name="s")

    @functools.partial(
        pl.kernel,
        mesh=mesh,
        out_type=jax.ShapeDtypeStruct((n_rows, H), jnp.float32),
        scratch_types=[
            pltpu.VMEM((4, _CHUNK), jnp.int32),
            pltpu.VMEM((4, _CHUNK, H), jnp.float32),
            pltpu.SemaphoreType.DMA,
        ],
        compiler_params=pltpu.CompilerParams(use_tc_tiling_on_sc=False),
    )
    def sc_fn(t_hbm, idx_hbm, out_hbm, idx_v, rows_v, sem):
        wid = lax.axis_index("s") * _NC + lax.axis_index("c")

        def chunk_body(ci, carry):
            g = wid * nch + ci
            pltpu.sync_copy(idx_hbm.at[g], idx_v)
            cps = [
                pltpu.async_copy(t_hbm.at[idx_v.at[k]], rows_v.at[k], sem)
                for k in range(4)
            ]
            for cp in cps:
                cp.wait()

            def row_body(r, c2):
                for v in range(vec):
                    sl = pl.ds(v * _LANES, _LANES)
                    acc = (rows_v[0, r, sl] + rows_v[1, r, sl]) + (
                        rows_v[2, r, sl] + rows_v[3, r, sl])
                    rows_v[0, r, sl] = jnp.maximum(acc, 0.0)
                return c2

            lax.fori_loop(0, _CHUNK, row_body, 0)
            pltpu.sync_copy(rows_v.at[0],
                            out_hbm.at[pl.ds(g * _CHUNK, _CHUNK)])
            return carry

        lax.fori_loop(0, nch, chunk_body, 0)

    return sc_fn


def kernel(pos_s, pos_e, pe_table, W, b):
    B, L = pos_s.shape
    P, H = pe_table.shape
    M = P // 2
    n = B * L * L
    T = _make_tables(pe_table, W, b)
    ps = pos_s.astype(jnp.int32)
    pe = pos_e.astype(jnp.int32)

    def rel(a, c, off):
        d = jnp.clip(a[:, :, None] - c[:, None, :] + M, 0, P - 1)
        return (d + off).reshape(-1)

    idx = jnp.stack([
        rel(ps, ps, 0),
        rel(ps, pe, P),
        rel(pe, ps, 2 * P),
        rel(pe, pe, 3 * P),
    ])
    idx_chunks = idx.reshape(4, n // _CHUNK, _CHUNK).transpose(1, 0, 2)
    out = _make_sc_lookup(n, H)(T, idx_chunks)
    return out.reshape(B, L, L, H)


# speedup vs baseline: 3.6757x; 1.1246x over previous
import functools

import jax
import jax.numpy as jnp
from jax import lax
from jax.experimental import pallas as pl
from jax.experimental.pallas import tpu as pltpu
from jax.experimental.pallas import tpu_sc as plsc

_NC, _NS = 2, 16
_NW = _NC * _NS
_CHUNK = 80
_LANES = 16


def _fuse_tables_body(pe_ref, a_ref, b_ref, out_ref):
    out_ref[0] = jnp.dot(pe_ref[...], a_ref[0],
                         preferred_element_type=jnp.float32) + b_ref[...]


def _make_tables(pe_table, W, b):
    P, H = pe_table.shape
    A = W.reshape(H, 4, H).transpose(1, 2, 0)
    bq = (0.25 * b).reshape(1, H).astype(jnp.float32)
    T = pl.pallas_call(
        _fuse_tables_body,
        grid=(4,),
        in_specs=[
            pl.BlockSpec((P, H), lambda k: (0, 0)),
            pl.BlockSpec((1, H, H), lambda k: (k, 0, 0)),
            pl.BlockSpec((1, H), lambda k: (0, 0)),
        ],
        out_specs=pl.BlockSpec((1, P, H), lambda k: (k, 0, 0)),
        out_shape=jax.ShapeDtypeStruct((4, P, H), jnp.float32),
    )(pe_table.astype(jnp.float32), A.astype(jnp.float32), bq)
    return T.reshape(4 * P, H)


def _make_sc_lookup(n_rows, H):
    per_w = n_rows // _NW
    nch = per_w // _CHUNK
    vec = H // _LANES
    mesh = plsc.VectorSubcoreMesh(core_axis_name="c", subcore_axis_name="s")

    @functools.partial(
        pl.kernel,
        mesh=mesh,
        out_type=jax.ShapeDtypeStruct((n_rows, H), jnp.float32),
        scratch_types=[
            pltpu.VMEM((2, 4, _CHUNK), jnp.int32),
            pltpu.VMEM((2, 4, _CHUNK, H), jnp.float32),
            pltpu.SemaphoreType.DMA,
            pltpu.SemaphoreType.DMA,
            pltpu.SemaphoreType.DMA,
            pltpu.SemaphoreType.DMA,
        ],
        compiler_params=pltpu.CompilerParams(use_tc_tiling_on_sc=False),
    )
    def sc_fn(t_hbm, idx_hbm, out_hbm, idx_v, rows_v, sg0, sg1, ss0, ss1):
        wid = lax.axis_index("s") * _NC + lax.axis_index("c")
        base = wid * nch
        sem_g = (sg0, sg1)
        sem_s = (ss0, ss1)

        def start_gathers(ci, buf):
            for k in range(4):
                pltpu.async_copy(t_hbm.at[idx_v.at[buf, k]],
                                 rows_v.at[buf, k], sem_g[buf])

        def wait_gathers(buf):
            pltpu.make_async_copy(out_hbm.at[pl.ds(0, 4 * _CHUNK)],
                                  rows_v.at[buf], sem_g[buf]).wait()

        def wait_scatter(buf):
            pltpu.make_async_copy(out_hbm.at[pl.ds(0, _CHUNK)],
                                  rows_v.at[buf, 0], sem_s[buf]).wait()

        def compute_and_scatter(ci, buf):
            def row_body(r, c2):
                for v in range(vec):
                    sl = pl.ds(v * _LANES, _LANES)
                    acc = (rows_v[buf, 0, r, sl] + rows_v[buf, 1, r, sl]) + (
                        rows_v[buf, 2, r, sl] + rows_v[buf, 3, r, sl])
                    rows_v[buf, 0, r, sl] = jnp.maximum(acc, 0.0)
                return c2

            lax.fori_loop(0, _CHUNK, row_body, 0)
            pltpu.async_copy(rows_v.at[buf, 0],
                             out_hbm.at[pl.ds((base + ci) * _CHUNK, _CHUNK)],
                             sem_s[buf])

        pltpu.sync_copy(idx_hbm.at[base], idx_v.at[0])
        start_gathers(0, 0)
        pltpu.sync_copy(idx_hbm.at[base + 1], idx_v.at[1])
        start_gathers(1, 1)
        wait_gathers(0)
        compute_and_scatter(0, 0)

        def chunk_body(h, carry):
            for par in range(2):
                ci = 1 + 2 * h + par
                buf = (1 + par) % 2
                nxt = 1 - buf

                @pl.when(ci + 1 < nch)
                def _():
                    pltpu.sync_copy(idx_hbm.at[base + ci + 1], idx_v.at[nxt])
                    wait_scatter(nxt)
                    start_gathers(ci + 1, nxt)

                wait_gathers(buf)
                compute_and_scatter(ci, buf)
            return carry

        lax.fori_loop(0, (nch - 1) // 2, chunk_body, 0)
        wait_scatter(0)
        wait_scatter(1)

    return sc_fn


def kernel(pos_s, pos_e, pe_table, W, b):
    B, L = pos_s.shape
    P, H = pe_table.shape
    M = P // 2
    n = B * L * L
    T = _make_tables(pe_table, W, b)
    ps = pos_s.astype(jnp.int32)
    pe = pos_e.astype(jnp.int32)

    def rel(a, c, off):
        d = jnp.clip(a[:, :, None] - c[:, None, :] + M, 0, P - 1)
        return (d + off).reshape(-1)

    idx = jnp.stack([
        rel(ps, ps, 0),
        rel(ps, pe, P),
        rel(pe, ps, 2 * P),
        rel(pe, pe, 3 * P),
    ])
    idx_chunks = idx.reshape(4, n // _CHUNK, _CHUNK).transpose(1, 0, 2)
    out = _make_sc_lookup(n, H)(T, idx_chunks)
    return out.reshape(B, L, L, H)
